# Initial kernel scaffold; baseline (speedup 1.0000x reference)
import jax
import jax.numpy as jnp
from jax.experimental import pallas as pl


def kernel(x, emb):
    raise NotImplementedError("write your pallas kernel here")


# speedup vs baseline: 1.6655x; 1.6655x over previous
import jax
import jax.numpy as jnp
from jax.experimental import pallas as pl


def _add_body(x_ref, emb_ref, out_ref):
    out_ref[0] = x_ref[0] + emb_ref[...]


def kernel(x, emb):
    batch, seq_len, d_model = x.shape
    blk = 1024
    while seq_len % blk:
        blk //= 2
    n_seq = seq_len // blk
    return pl.pallas_call(
        _add_body,
        grid=(n_seq, batch),
        in_specs=[
            pl.BlockSpec((1, blk, d_model), lambda s, b: (b, s, 0)),
            pl.BlockSpec((blk, d_model), lambda s, b: (s, 0)),
        ],
        out_specs=pl.BlockSpec((1, blk, d_model), lambda s, b: (b, s, 0)),
        out_shape=jax.ShapeDtypeStruct((batch, seq_len, d_model), x.dtype),
    )(x, emb[:seq_len])


# speedup vs baseline: 1.7369x; 1.0429x over previous
import jax
import jax.numpy as jnp
from jax.experimental import pallas as pl


def _add_body(x_ref, emb_ref, out_ref):
    out_ref[0] = x_ref[0] + emb_ref[...]


def kernel(x, emb):
    batch, seq_len, d_model = x.shape
    blk = 2048
    while seq_len % blk:
        blk //= 2
    n_seq = seq_len // blk
    return pl.pallas_call(
        _add_body,
        grid=(n_seq, batch),
        in_specs=[
            pl.BlockSpec((1, blk, d_model), lambda s, b: (b, s, 0)),
            pl.BlockSpec((blk, d_model), lambda s, b: (s, 0)),
        ],
        out_specs=pl.BlockSpec((1, blk, d_model), lambda s, b: (b, s, 0)),
        out_shape=jax.ShapeDtypeStruct((batch, seq_len, d_model), x.dtype),
    )(x, emb[:seq_len])
